# baseline (speedup 1.0000x reference)
import functools

import jax
import jax.numpy as jnp
from jax import lax
from jax.experimental import pallas as pl
from jax.experimental.pallas import tpu as pltpu
from jax.experimental.pallas import tpu_sc as plsc

_SHIFT = 44100
_B, _S, _C, _T = 8, 4, 2, 441000
_L = _T - _SHIFT
_NW = 32
_LT = (_L // 128) * 128
_LREM = _L - _LT
_M = 16000
_NFULL = _LT // _M
_TAILC = _LT - _NFULL * _M
_NCH = _NFULL + 1


def _rotate(a, b, s, idxvec, selmask):
    src = jnp.where(selmask, b, a)
    return jnp.take_along_axis(src, idxvec, axis=0, mode="promise_in_bounds")


def _sc_impl(wav3, offs):
    mesh = plsc.VectorSubcoreMesh(core_axis_name="c", subcore_axis_name="s")

    @functools.partial(
        pl.kernel,
        mesh=mesh,
        out_type=[
            jax.ShapeDtypeStruct((_NW, _C, _L), jnp.float32),
            jax.ShapeDtypeStruct((_NW, _C, _LT + 128), jnp.float32),
        ],
        scratch_types=[
            pltpu.VMEM((48,), jnp.int32),
            pltpu.VMEM((_C, _M + 128), jnp.float32),
            pltpu.VMEM((_C, _M + 128), jnp.float32),
            pltpu.VMEM((_C, _M), jnp.float32),
            pltpu.VMEM((_C, _M), jnp.float32),
            pltpu.VMEM((_C, 256), jnp.float32),
            pltpu.SemaphoreType.DMA,
            pltpu.SemaphoreType.DMA,
            pltpu.SemaphoreType.DMA,
            pltpu.SemaphoreType.DMA,
            pltpu.SemaphoreType.DMA,
            pltpu.SemaphoreType.DMA,
        ],
    )
    def k(wav_hbm, off_hbm, out_hbm, tails_hbm, off_v, r0, r1, t0, t1, traw,
          g0, g1, s0, s1, tg, ts):
        cid = lax.axis_index("c")
        sid = lax.axis_index("s")
        w = cid * 16 + sid

        pltpu.sync_copy(off_hbm.at[pl.ds(0, 32)], off_v.at[pl.ds(0, 32)])
        off = off_v[pl.ds(w, 16)][0]

        col0 = (off // 128) * 128
        phi = off - col0
        s = phi % 16
        phi16 = pl.multiple_of(phi - s, 16)
        lanes = lax.iota(jnp.int32, 16)
        idxvec = (lanes + s) & 15
        selmask = lanes < s

        raws = (r0, r1)
        stgs = (t0, t1)
        gsem = (g0, g1)
        ssem = (s0, s1)

        def chunk_cols(j):
            return _M if j < _NFULL else _TAILC

        def issue_gather(j):
            mlen = chunk_cols(j)
            return pltpu.async_copy(
                wav_hbm.at[w, :, pl.ds(pl.multiple_of(col0 + j * _M, 128),
                                       mlen + 128)],
                raws[j % 2].at[:, pl.ds(0, mlen + 128)], gsem[j % 2])

        def realign(src, dst, mlen):
            def body(i, carry):
                nxt = []
                for c in range(_C):
                    b = src[c, pl.ds(phi16 + i * 16 + 16, 16)]
                    dst[c, pl.ds(i * 16, 16)] = _rotate(carry[c], b, s,
                                                        idxvec, selmask)
                    nxt.append(b)
                return tuple(nxt)

            init = tuple(src[c, pl.ds(phi16, 16)] for c in range(_C))
            lax.fori_loop(0, mlen // 16, body, init)

        def issue_scatter(j):
            mlen = chunk_cols(j)
            return pltpu.async_copy(
                stgs[j % 2].at[:, pl.ds(0, mlen)],
                out_hbm.at[w, :, pl.ds(j * _M, mlen)], ssem[j % 2])

        tail_h = pltpu.async_copy(
            wav_hbm.at[w, :, pl.ds(pl.multiple_of(col0 + _LT, 128), 256)],
            traw, tg)

        gh = [None] * _NCH
        sh = [None] * _NCH
        gh[0] = issue_gather(0)
        for j in range(_NCH):
            gh[j].wait()
            if j + 1 < _NCH:
                gh[j + 1] = issue_gather(j + 1)
            if j - 2 >= 0:
                sh[j - 2].wait()
                sh[j - 2] = None
            realign(raws[j % 2], stgs[j % 2], chunk_cols(j))
            sh[j] = issue_scatter(j)

        tail_h.wait()
        realign(traw, traw, 128)
        pltpu.async_copy(traw.at[:, pl.ds(0, 128)],
                         tails_hbm.at[w, :, pl.ds(_LT, 128)], ts).wait()
        for h in sh:
            if h is not None:
                h.wait()

    return k(wav3, offs)


def _tc_patch(main, tails):
    def patch(main_any, tails_ref, out_ref):
        del main_any
        out_ref[...] = tails_ref[...]

    return pl.pallas_call(
        patch,
        grid=(1,),
        in_specs=[
            pl.BlockSpec(memory_space=pl.ANY),
            pl.BlockSpec((_NW, _C, 128), lambda i: (0, 0, _LT // 128)),
        ],
        out_specs=pl.BlockSpec((_NW, _C, 128), lambda i: (0, 0, _LT // 128)),
        out_shape=jax.ShapeDtypeStruct((_NW, _C, _L), jnp.float32),
        input_output_aliases={0: 0},
    )(main, tails)


def kernel(wav, offsets):
    wav3 = wav.reshape(_NW, _C, _T)
    offs = offsets.reshape(_NW).astype(jnp.int32)
    main, tails = _sc_impl(wav3, offs)
    out = _tc_patch(main, tails)
    return out.reshape(_B, _S, _C, _L)


# speedup vs baseline: 1.0004x; 1.0004x over previous
import functools

import jax
import jax.numpy as jnp
from jax import lax
from jax.experimental import pallas as pl
from jax.experimental.pallas import tpu as pltpu
from jax.experimental.pallas import tpu_sc as plsc

_SHIFT = 44100
_B, _S, _C, _T = 8, 4, 2, 441000
_L = _T - _SHIFT
_NW = 32
_LT = (_L // 128) * 128
_LREM = _L - _LT
_M = 16000
_NFULL = _LT // _M
_TAILC = _LT - _NFULL * _M
_NCH = _NFULL + 1


def _rotate(a, b, s, idxvec, selmask):
    src = jnp.where(selmask, b, a)
    return jnp.take_along_axis(src, idxvec, axis=0, mode="promise_in_bounds")


def _sc_impl(wav3, offs):
    mesh = plsc.VectorSubcoreMesh(core_axis_name="c", subcore_axis_name="s")

    @functools.partial(
        pl.kernel,
        mesh=mesh,
        out_type=[
            jax.ShapeDtypeStruct((_NW, _C, _L), jnp.float32),
            jax.ShapeDtypeStruct((_NW, _C, _LT + 128), jnp.float32),
        ],
        scratch_types=[
            pltpu.VMEM((48,), jnp.int32),
            pltpu.VMEM((_C, _M + 128), jnp.float32),
            pltpu.VMEM((_C, _M + 128), jnp.float32),
            pltpu.VMEM((_C, _M), jnp.float32),
            pltpu.VMEM((_C, _M), jnp.float32),
            pltpu.VMEM((_C, 256), jnp.float32),
            pltpu.SemaphoreType.DMA,
            pltpu.SemaphoreType.DMA,
            pltpu.SemaphoreType.DMA,
            pltpu.SemaphoreType.DMA,
            pltpu.SemaphoreType.DMA,
            pltpu.SemaphoreType.DMA,
        ],
    )
    def k(wav_hbm, off_hbm, out_hbm, tails_hbm, off_v, r0, r1, t0, t1, traw,
          g0, g1, s0, s1, tg, ts):
        cid = lax.axis_index("c")
        sid = lax.axis_index("s")
        w = cid * 16 + sid

        pltpu.sync_copy(off_hbm.at[pl.ds(0, 32)], off_v.at[pl.ds(0, 32)])
        off = off_v[pl.ds(w, 16)][0]

        col0 = (off // 128) * 128
        phi = off - col0
        s = phi % 16
        phi16 = pl.multiple_of(phi - s, 16)
        lanes = lax.iota(jnp.int32, 16)
        idxvec = (lanes + s) & 15
        selmask = lanes < s

        raws = (r0, r1)
        stgs = (t0, t1)
        gsem = (g0, g1)
        ssem = (s0, s1)

        def chunk_cols(j):
            return _M if j < _NFULL else _TAILC

        def issue_gather(j):
            mlen = chunk_cols(j)
            return pltpu.async_copy(
                wav_hbm.at[w, :, pl.ds(pl.multiple_of(col0 + j * _M, 128),
                                       mlen + 128)],
                raws[j % 2].at[:, pl.ds(0, mlen + 128)], gsem[j % 2])

        def realign(src, dst, mlen, inplace=False):
            def body(i, carry):
                nxt = []
                for c in range(_C):
                    b = src[c, pl.ds(phi16 + i * 16 + 16, 16)]
                    dst[c, pl.ds(i * 16, 16)] = _rotate(carry[c], b, s,
                                                        idxvec, selmask)
                    nxt.append(b)
                return tuple(nxt)

            init = tuple(src[c, pl.ds(phi16, 16)] for c in range(_C))
            if inplace:
                lax.fori_loop(0, mlen // 16, body, init)
            else:
                plsc.parallel_loop(0, mlen // 16, 1, unroll=8,
                                   carry=init)(body)

        def issue_scatter(j):
            mlen = chunk_cols(j)
            return pltpu.async_copy(
                stgs[j % 2].at[:, pl.ds(0, mlen)],
                out_hbm.at[w, :, pl.ds(j * _M, mlen)], ssem[j % 2])

        tail_h = pltpu.async_copy(
            wav_hbm.at[w, :, pl.ds(pl.multiple_of(col0 + _LT, 128), 256)],
            traw, tg)

        gh = [None] * _NCH
        sh = [None] * _NCH
        gh[0] = issue_gather(0)
        for j in range(_NCH):
            gh[j].wait()
            if j + 1 < _NCH:
                gh[j + 1] = issue_gather(j + 1)
            if j - 2 >= 0:
                sh[j - 2].wait()
                sh[j - 2] = None
            realign(raws[j % 2], stgs[j % 2], chunk_cols(j))
            sh[j] = issue_scatter(j)

        tail_h.wait()
        realign(traw, traw, 128, inplace=True)
        pltpu.async_copy(traw.at[:, pl.ds(0, 128)],
                         tails_hbm.at[w, :, pl.ds(_LT, 128)], ts).wait()
        for h in sh:
            if h is not None:
                h.wait()

    return k(wav3, offs)


def _tc_patch(main, tails):
    def patch(main_any, tails_ref, out_ref):
        del main_any
        out_ref[...] = tails_ref[...]

    return pl.pallas_call(
        patch,
        grid=(1,),
        in_specs=[
            pl.BlockSpec(memory_space=pl.ANY),
            pl.BlockSpec((_NW, _C, 128), lambda i: (0, 0, _LT // 128)),
        ],
        out_specs=pl.BlockSpec((_NW, _C, 128), lambda i: (0, 0, _LT // 128)),
        out_shape=jax.ShapeDtypeStruct((_NW, _C, _L), jnp.float32),
        input_output_aliases={0: 0},
    )(main, tails)


def kernel(wav, offsets):
    wav3 = wav.reshape(_NW, _C, _T)
    offs = offsets.reshape(_NW).astype(jnp.int32)
    main, tails = _sc_impl(wav3, offs)
    out = _tc_patch(main, tails)
    return out.reshape(_B, _S, _C, _L)


# speedup vs baseline: 1.6243x; 1.6237x over previous
import functools

import jax
import jax.numpy as jnp
from jax import lax
from jax.experimental import pallas as pl
from jax.experimental.pallas import tpu as pltpu
from jax.experimental.pallas import tpu_sc as plsc

_SHIFT = 44100
_B, _S, _C, _T = 8, 4, 2, 441000
_L = _T - _SHIFT
_NW = 32
_LT = (_L // 128) * 128
_LREM = _L - _LT
_M = 16000
_NFULL = _LT // _M
_TAILC = _LT - _NFULL * _M
_NCH = _NFULL + 1


def _rotate(a, b, s, idxvec, selmask):
    src = jnp.where(selmask, b, a)
    return jnp.take_along_axis(src, idxvec, axis=0, mode="promise_in_bounds")


def _sc_impl(wav3, offs):
    mesh = plsc.VectorSubcoreMesh(core_axis_name="c", subcore_axis_name="s")

    @functools.partial(
        pl.kernel,
        mesh=mesh,
        out_type=[
            jax.ShapeDtypeStruct((_NW, _C, _L), jnp.float32),
            jax.ShapeDtypeStruct((_NW, _C, _LT + 128), jnp.float32),
        ],
        scratch_types=[
            pltpu.VMEM((48,), jnp.int32),
            pltpu.VMEM((_C, _M + 128), jnp.float32),
            pltpu.VMEM((_C, _M + 128), jnp.float32),
            pltpu.VMEM((_C, _M), jnp.float32),
            pltpu.VMEM((_C, _M), jnp.float32),
            pltpu.VMEM((_C, 256), jnp.float32),
            pltpu.SemaphoreType.DMA,
            pltpu.SemaphoreType.DMA,
            pltpu.SemaphoreType.DMA,
            pltpu.SemaphoreType.DMA,
            pltpu.SemaphoreType.DMA,
            pltpu.SemaphoreType.DMA,
        ],
    )
    def k(wav_hbm, off_hbm, out_hbm, tails_hbm, off_v, r0, r1, t0, t1, traw,
          g0, g1, s0, s1, tg, ts):
        cid = lax.axis_index("c")
        sid = lax.axis_index("s")
        w = cid * 16 + sid

        pltpu.sync_copy(off_hbm.at[pl.ds(0, 32)], off_v.at[pl.ds(0, 32)])
        off = off_v[pl.ds(w, 16)][0]

        col0 = (off // 128) * 128
        phi = off - col0
        s = phi % 16
        phi16 = pl.multiple_of(phi - s, 16)
        lanes = lax.iota(jnp.int32, 16)
        idxvec = (lanes + s) & 15
        selmask = lanes < s

        raws = (r0, r1)
        stgs = (t0, t1)
        gsem = (g0, g1)
        ssem = (s0, s1)

        def chunk_cols(j):
            return _M if j < _NFULL else _TAILC

        def issue_gather(j):
            mlen = chunk_cols(j)
            return pltpu.async_copy(
                wav_hbm.at[w, :, pl.ds(pl.multiple_of(col0 + j * _M, 128),
                                       mlen + 128)],
                raws[j % 2].at[:, pl.ds(0, mlen + 128)], gsem[j % 2])

        def realign(src, dst, mlen, inplace=False):
            def body(i, carry):
                bs = [src[c, pl.ds(phi16 + i * 16 + 16, 16)]
                      for c in range(_C)]
                rs = [_rotate(carry[c], bs[c], s, idxvec, selmask)
                      for c in range(_C)]
                for c in range(_C):
                    dst[c, pl.ds(i * 16, 16)] = rs[c]
                return tuple(bs)

            init = tuple(src[c, pl.ds(phi16, 16)] for c in range(_C))
            if inplace:
                lax.fori_loop(0, mlen // 16, body, init)
            else:
                plsc.parallel_loop(0, mlen // 16, 1, unroll=8,
                                   carry=init)(body)

        def issue_scatter(j):
            mlen = chunk_cols(j)
            return pltpu.async_copy(
                stgs[j % 2].at[:, pl.ds(0, mlen)],
                out_hbm.at[w, :, pl.ds(j * _M, mlen)], ssem[j % 2])

        tail_h = pltpu.async_copy(
            wav_hbm.at[w, :, pl.ds(pl.multiple_of(col0 + _LT, 128), 256)],
            traw, tg)

        gh = [None] * _NCH
        sh = [None] * _NCH
        gh[0] = issue_gather(0)
        for j in range(_NCH):
            gh[j].wait()
            if j + 1 < _NCH:
                gh[j + 1] = issue_gather(j + 1)
            if j - 2 >= 0:
                sh[j - 2].wait()
                sh[j - 2] = None
            realign(raws[j % 2], stgs[j % 2], chunk_cols(j))
            sh[j] = issue_scatter(j)

        tail_h.wait()
        realign(traw, traw, 128, inplace=True)
        pltpu.async_copy(traw.at[:, pl.ds(0, 128)],
                         tails_hbm.at[w, :, pl.ds(_LT, 128)], ts).wait()
        for h in sh:
            if h is not None:
                h.wait()

    return k(wav3, offs)


def _tc_patch(main, tails):
    def patch(main_any, tails_ref, out_ref):
        del main_any
        out_ref[...] = tails_ref[...]

    return pl.pallas_call(
        patch,
        grid=(1,),
        in_specs=[
            pl.BlockSpec(memory_space=pl.ANY),
            pl.BlockSpec((_NW, _C, 128), lambda i: (0, 0, _LT // 128)),
        ],
        out_specs=pl.BlockSpec((_NW, _C, 128), lambda i: (0, 0, _LT // 128)),
        out_shape=jax.ShapeDtypeStruct((_NW, _C, _L), jnp.float32),
        input_output_aliases={0: 0},
    )(main, tails)


def kernel(wav, offsets):
    wav3 = wav.reshape(_NW, _C, _T)
    offs = offsets.reshape(_NW).astype(jnp.int32)
    main, tails = _sc_impl(wav3, offs)
    out = _tc_patch(main, tails)
    return out.reshape(_B, _S, _C, _L)


# speedup vs baseline: 2.3387x; 1.4398x over previous
import functools

import jax
import jax.numpy as jnp
from jax import lax
from jax.experimental import pallas as pl
from jax.experimental.pallas import tpu as pltpu
from jax.experimental.pallas import tpu_sc as plsc

_SHIFT = 44100
_B, _S, _C, _T = 8, 4, 2, 441000
_L = _T - _SHIFT
_NW = 32
_LT = (_L // 128) * 128
_LREM = _L - _LT
_M = 16000
_NFULL = _LT // _M
_TAILC = _LT - _NFULL * _M
_NCH = _NFULL + 1


def _rotate(a, b, s, idxvec, selmask):
    src = jnp.where(selmask, b, a)
    return jnp.take_along_axis(src, idxvec, axis=0, mode="promise_in_bounds")


def _sc_impl(wav3, offs):
    mesh = plsc.VectorSubcoreMesh(core_axis_name="c", subcore_axis_name="s")

    @functools.partial(
        pl.kernel,
        mesh=mesh,
        out_type=[
            jax.ShapeDtypeStruct((_NW, _C, _L), jnp.float32),
            jax.ShapeDtypeStruct((_NW, _C, _LT + 128), jnp.float32),
        ],
        scratch_types=[
            pltpu.VMEM((48,), jnp.int32),
            pltpu.VMEM((_C, _M + 128), jnp.float32),
            pltpu.VMEM((_C, _M + 128), jnp.float32),
            pltpu.VMEM((_C, _M), jnp.float32),
            pltpu.VMEM((_C, _M), jnp.float32),
            pltpu.VMEM((_C, 256), jnp.float32),
            pltpu.SemaphoreType.DMA,
            pltpu.SemaphoreType.DMA,
            pltpu.SemaphoreType.DMA,
            pltpu.SemaphoreType.DMA,
            pltpu.SemaphoreType.DMA,
            pltpu.SemaphoreType.DMA,
        ],
    )
    def k(wav_hbm, off_hbm, out_hbm, tails_hbm, off_v, r0, r1, t0, t1, traw,
          g0, g1, s0, s1, tg, ts):
        cid = lax.axis_index("c")
        sid = lax.axis_index("s")
        w = cid * 16 + sid

        pltpu.sync_copy(off_hbm.at[pl.ds(0, 32)], off_v.at[pl.ds(0, 32)])
        off = off_v[pl.ds(w, 16)][0]

        col0 = (off // 128) * 128
        phi = off - col0
        s = phi % 16
        phi16 = pl.multiple_of(phi - s, 16)
        lanes = lax.iota(jnp.int32, 16)
        idxvec = (lanes + s) & 15
        selmask = lanes < s

        raws = (r0, r1)
        stgs = (t0, t1)
        gsem = (g0, g1)
        ssem = (s0, s1)

        def chunk_cols(j):
            return _M if j < _NFULL else _TAILC

        def issue_gather(j):
            mlen = chunk_cols(j)
            return pltpu.async_copy(
                wav_hbm.at[w, :, pl.ds(pl.multiple_of(col0 + j * _M, 128),
                                       mlen + 128)],
                raws[j % 2].at[:, pl.ds(0, mlen + 128)], gsem[j % 2])

        def realign(src, dst, mlen, inplace=False):
            def body(i, carry):
                bs = [src[c, pl.ds(phi16 + i * 16 + 16, 16)]
                      for c in range(_C)]
                rs = [_rotate(carry[c], bs[c], s, idxvec, selmask)
                      for c in range(_C)]
                for c in range(_C):
                    dst[c, pl.ds(i * 16, 16)] = rs[c]
                return tuple(bs)

            def body2(i, carry):
                base = i * 32
                b0 = [src[c, pl.ds(phi16 + base + 16, 16)]
                      for c in range(_C)]
                b1 = [src[c, pl.ds(phi16 + base + 32, 16)]
                      for c in range(_C)]
                r0_ = [_rotate(carry[c], b0[c], s, idxvec, selmask)
                       for c in range(_C)]
                r1_ = [_rotate(b0[c], b1[c], s, idxvec, selmask)
                       for c in range(_C)]
                for c in range(_C):
                    dst[c, pl.ds(base, 16)] = r0_[c]
                    dst[c, pl.ds(base + 16, 16)] = r1_[c]
                return tuple(b1)

            init = tuple(src[c, pl.ds(phi16, 16)] for c in range(_C))
            if inplace:
                lax.fori_loop(0, mlen // 16, body, init)
            else:
                plsc.parallel_loop(0, mlen // 32, 1, unroll=4,
                                   carry=init)(body2)

        def issue_scatter(j):
            mlen = chunk_cols(j)
            return pltpu.async_copy(
                stgs[j % 2].at[:, pl.ds(0, mlen)],
                out_hbm.at[w, :, pl.ds(j * _M, mlen)], ssem[j % 2])

        tail_h = pltpu.async_copy(
            wav_hbm.at[w, :, pl.ds(pl.multiple_of(col0 + _LT, 128), 256)],
            traw, tg)

        gh = [None] * _NCH
        sh = [None] * _NCH
        gh[0] = issue_gather(0)
        for j in range(_NCH):
            gh[j].wait()
            if j + 1 < _NCH:
                gh[j + 1] = issue_gather(j + 1)
            if j - 2 >= 0:
                sh[j - 2].wait()
                sh[j - 2] = None
            realign(raws[j % 2], stgs[j % 2], chunk_cols(j))
            sh[j] = issue_scatter(j)

        tail_h.wait()
        realign(traw, traw, 128, inplace=True)
        pltpu.async_copy(traw.at[:, pl.ds(0, 128)],
                         tails_hbm.at[w, :, pl.ds(_LT, 128)], ts).wait()
        for h in sh:
            if h is not None:
                h.wait()

    return k(wav3, offs)


def _tc_patch(main, tails):
    def patch(main_any, tails_ref, out_ref):
        del main_any
        out_ref[...] = tails_ref[...]

    return pl.pallas_call(
        patch,
        grid=(1,),
        in_specs=[
            pl.BlockSpec(memory_space=pl.ANY),
            pl.BlockSpec((_NW, _C, 128), lambda i: (0, 0, _LT // 128)),
        ],
        out_specs=pl.BlockSpec((_NW, _C, 128), lambda i: (0, 0, _LT // 128)),
        out_shape=jax.ShapeDtypeStruct((_NW, _C, _L), jnp.float32),
        input_output_aliases={0: 0},
    )(main, tails)


def kernel(wav, offsets):
    wav3 = wav.reshape(_NW, _C, _T)
    offs = offsets.reshape(_NW).astype(jnp.int32)
    main, tails = _sc_impl(wav3, offs)
    out = _tc_patch(main, tails)
    return out.reshape(_B, _S, _C, _L)


# speedup vs baseline: 2.9633x; 1.2671x over previous
import functools

import jax
import jax.numpy as jnp
from jax import lax
from jax.experimental import pallas as pl
from jax.experimental.pallas import tpu as pltpu
from jax.experimental.pallas import tpu_sc as plsc

_SHIFT = 44100
_B, _S, _C, _T = 8, 4, 2, 441000
_L = _T - _SHIFT
_NW = 32
_LT = (_L // 128) * 128
_LREM = _L - _LT
_M = 16000
_NFULL = _LT // _M
_TAILC = _LT - _NFULL * _M
_NCH = _NFULL + 1


def _rotate(a, b, s, idxvec, selmask):
    src = jnp.where(selmask, b, a)
    return jnp.take_along_axis(src, idxvec, axis=0, mode="promise_in_bounds")


def _sc_impl(wav3, offs):
    mesh = plsc.VectorSubcoreMesh(core_axis_name="c", subcore_axis_name="s")

    @functools.partial(
        pl.kernel,
        mesh=mesh,
        out_type=[
            jax.ShapeDtypeStruct((_NW, _C, _L), jnp.float32),
            jax.ShapeDtypeStruct((_NW, _C, _LT + 128), jnp.float32),
        ],
        scratch_types=[
            pltpu.VMEM((48,), jnp.int32),
            pltpu.VMEM((_C, _M + 128), jnp.float32),
            pltpu.VMEM((_C, _M + 128), jnp.float32),
            pltpu.VMEM((_C, _M), jnp.float32),
            pltpu.VMEM((_C, _M), jnp.float32),
            pltpu.VMEM((_C, 256), jnp.float32),
            pltpu.SemaphoreType.DMA,
            pltpu.SemaphoreType.DMA,
            pltpu.SemaphoreType.DMA,
            pltpu.SemaphoreType.DMA,
            pltpu.SemaphoreType.DMA,
            pltpu.SemaphoreType.DMA,
        ],
    )
    def k(wav_hbm, off_hbm, out_hbm, tails_hbm, off_v, r0, r1, t0, t1, traw,
          g0, g1, s0, s1, tg, ts):
        cid = lax.axis_index("c")
        sid = lax.axis_index("s")
        w = cid * 16 + sid

        pltpu.sync_copy(off_hbm.at[pl.ds(0, 32)], off_v.at[pl.ds(0, 32)])
        off = off_v[pl.ds(w, 16)][0]

        col0 = (off // 128) * 128
        phi = off - col0
        s = phi % 16
        phi16 = pl.multiple_of(phi - s, 16)
        lanes = lax.iota(jnp.int32, 16)
        idxvec = (lanes + s) & 15
        selmask = lanes < s

        raws = (r0, r1)
        stgs = (t0, t1)
        gsem = (g0, g1)
        ssem = (s0, s1)

        def chunk_cols(j):
            return _M if j < _NFULL else _TAILC

        def issue_gather(j):
            mlen = chunk_cols(j)
            return pltpu.async_copy(
                wav_hbm.at[w, :, pl.ds(pl.multiple_of(col0 + j * _M, 128),
                                       mlen + 128)],
                raws[j % 2].at[:, pl.ds(0, mlen + 128)], gsem[j % 2])

        def realign(src, dst, mlen, inplace=False):
            def body(i, carry):
                bs = [src[c, pl.ds(phi16 + i * 16 + 16, 16)]
                      for c in range(_C)]
                rs = [_rotate(carry[c], bs[c], s, idxvec, selmask)
                      for c in range(_C)]
                for c in range(_C):
                    dst[c, pl.ds(i * 16, 16)] = rs[c]
                return tuple(bs)

            U = 4

            def body2(i, carry):
                base = i * (16 * U)
                bs = [[src[c, pl.ds(phi16 + base + 16 * (u + 1), 16)]
                       for c in range(_C)] for u in range(U)]
                prev = carry
                outs = []
                for u in range(U):
                    outs.append([_rotate(prev[c], bs[u][c], s, idxvec,
                                         selmask) for c in range(_C)])
                    prev = bs[u]
                for u in range(U):
                    for c in range(_C):
                        dst[c, pl.ds(base + 16 * u, 16)] = outs[u][c]
                return tuple(bs[U - 1])

            init = tuple(src[c, pl.ds(phi16, 16)] for c in range(_C))
            if inplace:
                lax.fori_loop(0, mlen // 16, body, init)
            else:
                plsc.parallel_loop(0, mlen // (16 * U), 1, unroll=2,
                                   carry=init)(body2)

        def issue_scatter(j):
            mlen = chunk_cols(j)
            return pltpu.async_copy(
                stgs[j % 2].at[:, pl.ds(0, mlen)],
                out_hbm.at[w, :, pl.ds(j * _M, mlen)], ssem[j % 2])

        tail_h = pltpu.async_copy(
            wav_hbm.at[w, :, pl.ds(pl.multiple_of(col0 + _LT, 128), 256)],
            traw, tg)

        gh = [None] * _NCH
        sh = [None] * _NCH
        gh[0] = issue_gather(0)
        for j in range(_NCH):
            gh[j].wait()
            if j + 1 < _NCH:
                gh[j + 1] = issue_gather(j + 1)
            if j - 2 >= 0:
                sh[j - 2].wait()
                sh[j - 2] = None
            realign(raws[j % 2], stgs[j % 2], chunk_cols(j))
            sh[j] = issue_scatter(j)

        tail_h.wait()
        realign(traw, traw, 128, inplace=True)
        pltpu.async_copy(traw.at[:, pl.ds(0, 128)],
                         tails_hbm.at[w, :, pl.ds(_LT, 128)], ts).wait()
        for h in sh:
            if h is not None:
                h.wait()

    return k(wav3, offs)


def _tc_patch(main, tails):
    def patch(main_any, tails_ref, out_ref):
        del main_any
        out_ref[...] = tails_ref[...]

    return pl.pallas_call(
        patch,
        grid=(1,),
        in_specs=[
            pl.BlockSpec(memory_space=pl.ANY),
            pl.BlockSpec((_NW, _C, 128), lambda i: (0, 0, _LT // 128)),
        ],
        out_specs=pl.BlockSpec((_NW, _C, 128), lambda i: (0, 0, _LT // 128)),
        out_shape=jax.ShapeDtypeStruct((_NW, _C, _L), jnp.float32),
        input_output_aliases={0: 0},
    )(main, tails)


def kernel(wav, offsets):
    wav3 = wav.reshape(_NW, _C, _T)
    offs = offsets.reshape(_NW).astype(jnp.int32)
    main, tails = _sc_impl(wav3, offs)
    out = _tc_patch(main, tails)
    return out.reshape(_B, _S, _C, _L)


# speedup vs baseline: 2.9834x; 1.0068x over previous
import functools

import jax
import jax.numpy as jnp
from jax import lax
from jax.experimental import pallas as pl
from jax.experimental.pallas import tpu as pltpu
from jax.experimental.pallas import tpu_sc as plsc

_SHIFT = 44100
_B, _S, _C, _T = 8, 4, 2, 441000
_L = _T - _SHIFT
_NW = 32
_LT = (_L // 128) * 128
_LREM = _L - _LT
_M = 16000
_NFULL = _LT // _M
_TAILC = _LT - _NFULL * _M
_NCH = _NFULL + 1


def _rotate(a, b, s, idxvec, selmask):
    src = jnp.where(selmask, b, a)
    return jnp.take_along_axis(src, idxvec, axis=0, mode="promise_in_bounds")


def _sc_impl(wav3, offs):
    mesh = plsc.VectorSubcoreMesh(core_axis_name="c", subcore_axis_name="s")

    @functools.partial(
        pl.kernel,
        mesh=mesh,
        out_type=[
            jax.ShapeDtypeStruct((_NW, _C, _L), jnp.float32),
            jax.ShapeDtypeStruct((_NW, _C, _LT + 128), jnp.float32),
        ],
        scratch_types=[
            pltpu.VMEM((48,), jnp.int32),
            pltpu.VMEM((_C, _M + 128), jnp.float32),
            pltpu.VMEM((_C, _M + 128), jnp.float32),
            pltpu.VMEM((_C, _M), jnp.float32),
            pltpu.VMEM((_C, _M), jnp.float32),
            pltpu.VMEM((_C, 256), jnp.float32),
            pltpu.SemaphoreType.DMA,
            pltpu.SemaphoreType.DMA,
            pltpu.SemaphoreType.DMA,
            pltpu.SemaphoreType.DMA,
            pltpu.SemaphoreType.DMA,
            pltpu.SemaphoreType.DMA,
        ],
    )
    def k(wav_hbm, off_hbm, out_hbm, tails_hbm, off_v, r0, r1, t0, t1, traw,
          g0, g1, s0, s1, tg, ts):
        cid = lax.axis_index("c")
        sid = lax.axis_index("s")
        w = cid * 16 + sid

        pltpu.sync_copy(off_hbm.at[pl.ds(0, 32)], off_v.at[pl.ds(0, 32)])
        off = off_v[pl.ds(w, 16)][0]

        col0 = (off // 128) * 128
        phi = off - col0
        s = phi % 16
        phi16 = pl.multiple_of(phi - s, 16)
        lanes = lax.iota(jnp.int32, 16)
        idxvec = (lanes + s) & 15
        selmask = lanes < s

        raws = (r0, r1)
        stgs = (t0, t1)
        gsem = (g0, g1)
        ssem = (s0, s1)

        def chunk_cols(j):
            return _M if j < _NFULL else _TAILC

        def issue_gather(j):
            mlen = chunk_cols(j)
            return pltpu.async_copy(
                wav_hbm.at[w, :, pl.ds(pl.multiple_of(col0 + j * _M, 128),
                                       mlen + 128)],
                raws[j % 2].at[:, pl.ds(0, mlen + 128)], gsem[j % 2])

        def realign(src, dst, mlen, inplace=False):
            def body(i, carry):
                bs = [src[c, pl.ds(phi16 + i * 16 + 16, 16)]
                      for c in range(_C)]
                rs = [_rotate(carry[c], bs[c], s, idxvec, selmask)
                      for c in range(_C)]
                for c in range(_C):
                    dst[c, pl.ds(i * 16, 16)] = rs[c]
                return tuple(bs)

            U = 8

            def body2(i, carry):
                base = i * (16 * U)
                bs = [[src[c, pl.ds(phi16 + base + 16 * (u + 1), 16)]
                       for c in range(_C)] for u in range(U)]
                prev = carry
                outs = []
                for u in range(U):
                    outs.append([_rotate(prev[c], bs[u][c], s, idxvec,
                                         selmask) for c in range(_C)])
                    prev = bs[u]
                for u in range(U):
                    for c in range(_C):
                        dst[c, pl.ds(base + 16 * u, 16)] = outs[u][c]
                return tuple(bs[U - 1])

            init = tuple(src[c, pl.ds(phi16, 16)] for c in range(_C))
            if inplace:
                lax.fori_loop(0, mlen // 16, body, init)
            else:
                plsc.parallel_loop(0, mlen // (16 * U), 1, unroll=1,
                                   carry=init)(body2)

        def issue_scatter(j):
            mlen = chunk_cols(j)
            return pltpu.async_copy(
                stgs[j % 2].at[:, pl.ds(0, mlen)],
                out_hbm.at[w, :, pl.ds(j * _M, mlen)], ssem[j % 2])

        tail_h = pltpu.async_copy(
            wav_hbm.at[w, :, pl.ds(pl.multiple_of(col0 + _LT, 128), 256)],
            traw, tg)

        gh = [None] * _NCH
        sh = [None] * _NCH
        gh[0] = issue_gather(0)
        for j in range(_NCH):
            gh[j].wait()
            if j + 1 < _NCH:
                gh[j + 1] = issue_gather(j + 1)
            if j - 2 >= 0:
                sh[j - 2].wait()
                sh[j - 2] = None
            realign(raws[j % 2], stgs[j % 2], chunk_cols(j))
            sh[j] = issue_scatter(j)

        tail_h.wait()
        realign(traw, traw, 128, inplace=True)
        pltpu.async_copy(traw.at[:, pl.ds(0, 128)],
                         tails_hbm.at[w, :, pl.ds(_LT, 128)], ts).wait()
        for h in sh:
            if h is not None:
                h.wait()

    return k(wav3, offs)


def _tc_patch(main, tails):
    def patch(main_any, tails_ref, out_ref):
        del main_any
        out_ref[...] = tails_ref[...]

    return pl.pallas_call(
        patch,
        grid=(1,),
        in_specs=[
            pl.BlockSpec(memory_space=pl.ANY),
            pl.BlockSpec((_NW, _C, 128), lambda i: (0, 0, _LT // 128)),
        ],
        out_specs=pl.BlockSpec((_NW, _C, 128), lambda i: (0, 0, _LT // 128)),
        out_shape=jax.ShapeDtypeStruct((_NW, _C, _L), jnp.float32),
        input_output_aliases={0: 0},
    )(main, tails)


def kernel(wav, offsets):
    wav3 = wav.reshape(_NW, _C, _T)
    offs = offsets.reshape(_NW).astype(jnp.int32)
    main, tails = _sc_impl(wav3, offs)
    out = _tc_patch(main, tails)
    return out.reshape(_B, _S, _C, _L)
